# baseline (speedup 1.0000x reference)
import functools

import jax
import jax.numpy as jnp
from jax import lax
from jax.experimental import pallas as pl
from jax.experimental.pallas import tpu as pltpu
from jax.experimental.pallas import tpu_sc as plsc

BT = 128
RB = 512


def _router_body(x_ref, wr_ref, sel_ref, w_ref):
    E = wr_ref.shape[0]
    x = x_ref[...]
    wr = wr_ref[...]
    logits = lax.dot_general(x, wr, (((1,), (1,)), ((), ())),
                             preferred_element_type=jnp.float32)
    m = jnp.max(logits, axis=1, keepdims=True)
    s = jnp.sum(jnp.exp(logits - m), axis=1)
    eidx = lax.broadcasted_iota(jnp.int32, logits.shape, 1)
    sel = jnp.min(jnp.where(logits == m, eidx, E), axis=1)
    sel_ref[...] = sel.reshape(1, -1)
    w_ref[...] = (1.0 / s).reshape(1, -1)


def _route(xs, W_router):
    T, D = xs.shape
    E = W_router.shape[0]
    nblk = T // RB
    sel, w = pl.pallas_call(
        _router_body,
        grid=(nblk,),
        in_specs=[
            pl.BlockSpec((RB, D), lambda j: (j, 0)),
            pl.BlockSpec((E, D), lambda j: (0, 0)),
        ],
        out_specs=[
            pl.BlockSpec((1, RB), lambda j: (0, j)),
            pl.BlockSpec((1, RB), lambda j: (0, j)),
        ],
        out_shape=[
            jax.ShapeDtypeStruct((1, T), jnp.int32),
            jax.ShapeDtypeStruct((1, T), jnp.float32),
        ],
    )(xs, W_router)
    return sel[0], w[0]


def _gmm_body(be_ref, meta_ref, x_ref, wfc_ref, bfc_ref, wproj_ref,
              bproj_ref, wgt_ref, y_ref):
    j = pl.program_id(0)

    @pl.when(j < meta_ref[0])
    def _():
        x = x_ref[...]
        h = lax.dot_general(x, wfc_ref[0], (((1,), (1,)), ((), ())),
                            preferred_element_type=jnp.float32)
        h = h + bfc_ref[0]
        h = 0.5 * h * (1.0 + lax.erf(h * 0.7071067811865476))
        y = lax.dot_general(h, wproj_ref[0], (((1,), (1,)), ((), ())),
                            preferred_element_type=jnp.float32)
        y = y + bproj_ref[0]
        y_ref[...] = y * wgt_ref[...]


def _gmm(xs_pad, W_fc, b_fc, W_proj, b_proj, w_pad, be, meta, nb):
    TP, D = xs_pad.shape
    E, F, _ = W_fc.shape
    grid_spec = pltpu.PrefetchScalarGridSpec(
        num_scalar_prefetch=2,
        grid=(nb,),
        in_specs=[
            pl.BlockSpec((BT, D), lambda j, be, meta: (j, 0)),
            pl.BlockSpec((1, F, D), lambda j, be, meta: (be[j], 0, 0)),
            pl.BlockSpec((1, 1, F), lambda j, be, meta: (be[j], 0, 0)),
            pl.BlockSpec((1, D, F), lambda j, be, meta: (be[j], 0, 0)),
            pl.BlockSpec((1, 1, D), lambda j, be, meta: (be[j], 0, 0)),
            pl.BlockSpec((BT, 1), lambda j, be, meta: (j, 0)),
        ],
        out_specs=pl.BlockSpec((BT, D), lambda j, be, meta: (j, 0)),
    )
    return pl.pallas_call(
        _gmm_body,
        grid_spec=grid_spec,
        out_shape=jax.ShapeDtypeStruct((TP, D), jnp.float32),
        compiler_params=pltpu.CompilerParams(
            dimension_semantics=("arbitrary",)),
    )(be, meta, xs_pad, W_fc, b_fc.reshape(E, 1, F), W_proj,
      b_proj.reshape(E, 1, D), w_pad)


def _sc_gather(table, idx, chunk=128):
    n_rows, d = table.shape
    n_idx = idx.shape[0]
    info = plsc.get_sparse_core_info()
    nw = info.num_cores * info.num_subcores
    per_w = n_idx // nw
    assert per_w * nw == n_idx and per_w % chunk == 0
    n_ch = per_w // chunk
    mesh = plsc.VectorSubcoreMesh(core_axis_name="c", subcore_axis_name="s")

    @functools.partial(
        pl.kernel, mesh=mesh,
        out_type=jax.ShapeDtypeStruct((n_idx, d), jnp.float32),
        scratch_types=[
            pltpu.VMEM((chunk,), jnp.int32),
            pltpu.VMEM((chunk, d), jnp.float32),
            pltpu.SemaphoreType.DMA,
        ],
    )
    def k(table_hbm, idx_hbm, out_hbm, idx_v, rows_v, sem):
        wid = lax.axis_index("s") * info.num_cores + lax.axis_index("c")
        base = wid * per_w
        for c in range(n_ch):
            off = base + c * chunk
            pltpu.sync_copy(idx_hbm.at[pl.ds(off, chunk)], idx_v)
            pltpu.async_copy(table_hbm.at[idx_v], rows_v, sem).wait()
            pltpu.sync_copy(rows_v, out_hbm.at[pl.ds(off, chunk)])

    return k(table, idx)


def kernel(x, W_router, W_fc, b_fc, W_proj, b_proj):
    B, S, D = x.shape
    E, F, _ = W_fc.shape
    T = B * S
    NB = T // BT + E
    TP = NB * BT

    xs = x.reshape(T, D)
    sel, w = _route(xs, W_router)

    order = jnp.argsort(sel).astype(jnp.int32)
    sorted_sel = sel[order]
    counts = jnp.zeros((E,), jnp.int32).at[sel].add(1)
    nblk = (counts + BT - 1) // BT
    blk_end = jnp.cumsum(nblk).astype(jnp.int32)
    blk_start = jnp.concatenate([jnp.zeros((1,), jnp.int32), blk_end[:-1]])
    nba = blk_end[E - 1]
    offs_raw = jnp.concatenate(
        [jnp.zeros((1,), jnp.int32), jnp.cumsum(counts).astype(jnp.int32)[:-1]])
    ranks = jnp.arange(T, dtype=jnp.int32) - offs_raw[sorted_sel]
    pos_sorted = blk_start[sorted_sel] * BT + ranks
    tok_idx = (jnp.arange(TP, dtype=jnp.int32) % T).at[pos_sorted].set(order)
    pos = jnp.zeros((T,), jnp.int32).at[order].set(pos_sorted)
    jblk = jnp.minimum(jnp.arange(NB, dtype=jnp.int32), nba - 1)
    block_expert = jnp.searchsorted(blk_end, jblk, side="right").astype(jnp.int32)
    meta = nba.reshape(1)
    w_pad = w[tok_idx].reshape(TP, 1)

    xs_pad = _sc_gather(xs, tok_idx)

    y_pad = _gmm(xs_pad, W_fc, b_fc, W_proj, b_proj, w_pad,
                 block_expert, meta, NB)

    out = _sc_gather(y_pad, pos)
    return out.reshape(B, S, D)


# speedup vs baseline: 1.2130x; 1.2130x over previous
import functools

import jax
import jax.numpy as jnp
from jax import lax
from jax.experimental import pallas as pl
from jax.experimental.pallas import tpu as pltpu
from jax.experimental.pallas import tpu_sc as plsc

BT = 128
RB = 512
CH = 128
SPREAD = 1024
L = 16


def _sc_mesh():
    return plsc.VectorSubcoreMesh(core_axis_name="c", subcore_axis_name="s")


def _wid():
    info = plsc.get_sparse_core_info()
    return lax.axis_index("s") * info.num_cores + lax.axis_index("c")


def _nw():
    info = plsc.get_sparse_core_info()
    return info.num_cores * info.num_subcores


def _lane_extract(vec, k):
    lanes = lax.iota(jnp.int32, L)
    return jnp.sum(jnp.where(lanes == k, vec, 0))


def _router_body(x_ref, wr_ref, sel_ref, w_ref):
    E = wr_ref.shape[0]
    x = x_ref[...]
    wr = wr_ref[...]
    logits = lax.dot_general(x, wr, (((1,), (1,)), ((), ())),
                             preferred_element_type=jnp.float32)
    m = jnp.max(logits, axis=1, keepdims=True)
    s = jnp.sum(jnp.exp(logits - m), axis=1)
    eidx = lax.broadcasted_iota(jnp.int32, logits.shape, 1)
    sel = jnp.min(jnp.where(logits == m, eidx, E), axis=1)
    sel_ref[...] = sel.reshape(1, -1)
    w_ref[...] = (1.0 / s).reshape(1, -1)


def _route(xs, W_router):
    T, D = xs.shape
    E = W_router.shape[0]
    nblk = T // RB
    sel, w = pl.pallas_call(
        _router_body,
        grid=(nblk,),
        in_specs=[
            pl.BlockSpec((RB, D), lambda j: (j, 0)),
            pl.BlockSpec((E, D), lambda j: (0, 0)),
        ],
        out_specs=[
            pl.BlockSpec((1, RB), lambda j: (0, j)),
            pl.BlockSpec((1, RB), lambda j: (0, j)),
        ],
        out_shape=[
            jax.ShapeDtypeStruct((1, T), jnp.int32),
            jax.ShapeDtypeStruct((1, T), jnp.float32),
        ],
    )(xs, W_router)
    return sel[0], w[0]


def _sc_count(sel, E):
    T = sel.shape[0]
    NW = _nw()
    EPW = E // NW
    n_vec = T // L

    @functools.partial(
        pl.kernel, mesh=_sc_mesh(),
        out_type=jax.ShapeDtypeStruct((NW, 8), jnp.int32),
        scratch_types=[
            pltpu.VMEM((T,), jnp.int32),
            pltpu.VMEM((L,), jnp.int32),
        ],
        compiler_params=pltpu.CompilerParams(needs_layout_passes=False),
    )
    def k(sel_hbm, cnt_hbm, sel_v, cv_v):
        wid = _wid()
        pltpu.sync_copy(sel_hbm, sel_v)
        lanes = lax.iota(jnp.int32, L)

        def body(i, carry):
            v = sel_v[pl.ds(i * L, L)]
            return tuple(carry[j] + jnp.sum(jnp.where(v == wid * EPW + j, 1, 0))
                         for j in range(EPW))

        cs = lax.fori_loop(0, n_vec, body, (0,) * EPW)
        cv = jnp.zeros((L,), jnp.int32)
        for j in range(EPW):
            cv = jnp.where(lanes == j, cs[j], cv)
        cv_v[...] = cv
        pltpu.sync_copy(cv_v.at[pl.ds(0, 8)], cnt_hbm.at[wid])

    return k(sel)


def _sc_fill(sel, base, E, TP):
    T = sel.shape[0]
    NW = _nw()
    EPW = E // NW
    n_vec = T // L
    maxb = T // BT

    @functools.partial(
        pl.kernel, mesh=_sc_mesh(),
        out_type=jax.ShapeDtypeStruct((TP,), jnp.int32),
        scratch_types=[
            pltpu.VMEM((T,), jnp.int32),
            pltpu.VMEM((T,), jnp.int32),
            pltpu.VMEM((L,), jnp.int32),
        ],
        compiler_params=pltpu.CompilerParams(needs_layout_passes=False),
    )
    def k(sel_hbm, base_hbm, tok_hbm, sel_v, buf, base_v):
        wid = _wid()
        pltpu.sync_copy(sel_hbm, sel_v)
        pltpu.sync_copy(base_hbm.at[pl.ds(wid * 8, 8)], base_v.at[pl.ds(0, 8)])
        lanes = lax.iota(jnp.int32, L)
        bvec = base_v[...]
        for j in range(EPW):
            e = wid * EPW + j

            def pre(i, _):
                buf[pl.ds(i * L, L)] = T + ((i * L + lanes) & (SPREAD - 1))
                return 0

            lax.fori_loop(0, n_vec, pre, 0)

            def scan(i, off):
                v = sel_v[pl.ds(i * L, L)]
                m = v == e
                mi = jnp.where(m, 1, 0)
                cs = plsc.cumsum(mi)
                plsc.store_scatter(buf, [off + cs - mi], i * L + lanes,
                                   mask=m)
                return off + _lane_extract(cs, L - 1)

            cnt = lax.fori_loop(0, n_vec, scan, 0)
            base_blk = _lane_extract(bvec, j)
            nb = (cnt + BT - 1) // BT
            for b in range(maxb):
                @pl.when(b < nb)
                def _():
                    pltpu.sync_copy(
                        buf.at[pl.ds(b * BT, BT)],
                        tok_hbm.at[pl.ds((base_blk + b) * BT, BT)])

    return k(sel, base)


def _sc_dispatch(xs, tok_idx, aux):
    T, D = xs.shape
    TP = tok_idx.shape[0]
    NW = _nw()
    per_w = TP // NW
    n_ch = per_w // CH

    @functools.partial(
        pl.kernel, mesh=_sc_mesh(),
        out_type=jax.ShapeDtypeStruct((TP, D), jnp.float32),
        scratch_types=[
            pltpu.VMEM((CH,), jnp.int32),
            pltpu.VMEM((CH,), jnp.int32),
            pltpu.VMEM((CH, D), jnp.float32),
            pltpu.VMEM((L,), jnp.int32),
            pltpu.SemaphoreType.DMA,
        ],
        compiler_params=pltpu.CompilerParams(needs_layout_passes=False),
    )
    def k(xs_hbm, idx_hbm, aux_hbm, out_hbm, idx_v, idx2_v, rows_v, lim_v, sem):
        wid = _wid()
        base = wid * per_w
        pltpu.sync_copy(aux_hbm, lim_v.at[pl.ds(0, 8)])
        limit = _lane_extract(lim_v[...], 0)
        for c in range(n_ch):
            off = base + c * CH

            @pl.when(off < limit)
            def _():
                pltpu.sync_copy(idx_hbm.at[pl.ds(off, CH)], idx_v)
                for u in range(CH // L):
                    idx2_v[pl.ds(u * L, L)] = idx_v[pl.ds(u * L, L)] & (T - 1)
                pltpu.async_copy(xs_hbm.at[idx2_v], rows_v, sem).wait()
                pltpu.sync_copy(rows_v, out_hbm.at[pl.ds(off, CH)])

    return k(xs, tok_idx, aux)


def _sc_combine(y_pad, tok_idx, aux, T):
    TP, D = y_pad.shape
    NW = _nw()
    per_w = TP // NW
    n_ch = per_w // CH

    @functools.partial(
        pl.kernel, mesh=_sc_mesh(),
        out_type=jax.ShapeDtypeStruct((T + SPREAD, D), jnp.float32),
        scratch_types=[
            pltpu.VMEM((CH,), jnp.int32),
            pltpu.VMEM((CH, D), jnp.float32),
            pltpu.VMEM((L,), jnp.int32),
            pltpu.SemaphoreType.DMA,
        ],
        compiler_params=pltpu.CompilerParams(needs_layout_passes=False),
    )
    def k(y_hbm, idx_hbm, aux_hbm, out_hbm, idx_v, rows_v, lim_v, sem):
        wid = _wid()
        base = wid * per_w
        pltpu.sync_copy(aux_hbm, lim_v.at[pl.ds(0, 8)])
        limit = _lane_extract(lim_v[...], 0)
        for c in range(n_ch):
            off = base + c * CH

            @pl.when(off < limit)
            def _():
                pltpu.sync_copy(idx_hbm.at[pl.ds(off, CH)], idx_v)
                pltpu.sync_copy(y_hbm.at[pl.ds(off, CH)], rows_v)
                pltpu.async_copy(rows_v, out_hbm.at[idx_v], sem).wait()

    return k(y_pad, tok_idx, aux)


def _gmm_body(be_ref, meta_ref, x_ref, wfc_ref, bfc_ref, wproj_ref,
              bproj_ref, wgt_ref, y_ref):
    j = pl.program_id(0)

    @pl.when(j < meta_ref[0])
    def _():
        x = x_ref[...]
        h = lax.dot_general(x, wfc_ref[0], (((1,), (1,)), ((), ())),
                            preferred_element_type=jnp.float32)
        h = h + bfc_ref[0]
        h = 0.5 * h * (1.0 + lax.erf(h * 0.7071067811865476))
        y = lax.dot_general(h, wproj_ref[0], (((1,), (1,)), ((), ())),
                            preferred_element_type=jnp.float32)
        y = y + bproj_ref[0]
        y_ref[...] = y * wgt_ref[...]


def _gmm(xs_pad, W_fc, b_fc, W_proj, b_proj, w_pad, be, meta, nb):
    TP, D = xs_pad.shape
    E, F, _ = W_fc.shape
    grid_spec = pltpu.PrefetchScalarGridSpec(
        num_scalar_prefetch=2,
        grid=(nb,),
        in_specs=[
            pl.BlockSpec((BT, D), lambda j, be, meta: (j, 0)),
            pl.BlockSpec((1, F, D), lambda j, be, meta: (be[j], 0, 0)),
            pl.BlockSpec((1, 1, F), lambda j, be, meta: (be[j], 0, 0)),
            pl.BlockSpec((1, D, F), lambda j, be, meta: (be[j], 0, 0)),
            pl.BlockSpec((1, 1, D), lambda j, be, meta: (be[j], 0, 0)),
            pl.BlockSpec((BT, 1), lambda j, be, meta: (j, 0)),
        ],
        out_specs=pl.BlockSpec((BT, D), lambda j, be, meta: (j, 0)),
    )
    return pl.pallas_call(
        _gmm_body,
        grid_spec=grid_spec,
        out_shape=jax.ShapeDtypeStruct((TP, D), jnp.float32),
        compiler_params=pltpu.CompilerParams(
            dimension_semantics=("arbitrary",)),
    )(be, meta, xs_pad, W_fc, b_fc.reshape(E, 1, F), W_proj,
      b_proj.reshape(E, 1, D), w_pad)


def kernel(x, W_router, W_fc, b_fc, W_proj, b_proj):
    B, S, D = x.shape
    E, F, _ = W_fc.shape
    T = B * S
    NB = T // BT + E
    TP = NB * BT
    NW = 32
    EPW = E // NW

    xs = x.reshape(T, D)
    sel, w = _route(xs, W_router)

    counts8 = jnp.pad(jnp.zeros((E,), jnp.int32).at[sel].add(1).reshape(NW, EPW), ((0, 0), (0, 8 - EPW)))
    counts = counts8[:, :EPW].reshape(E)
    nblk = (counts + BT - 1) // BT
    blk_end = jnp.cumsum(nblk).astype(jnp.int32)
    nba = blk_end[E - 1]
    blk_start = blk_end - nblk
    base8 = jnp.zeros((NW, 8), jnp.int32).at[:, :EPW].set(
        blk_start.reshape(NW, EPW)).reshape(NW * 8)
    aux = jnp.full((8,), nba * BT, jnp.int32)
    jblk = jnp.minimum(jnp.arange(NB, dtype=jnp.int32), nba - 1)
    block_expert = jnp.searchsorted(blk_end, jblk, side="right").astype(jnp.int32)
    meta = nba.reshape(1)

    tok_idx = _sc_fill(sel, base8, E, TP)
    w_pad = w[jnp.clip(tok_idx, 0, T - 1)].reshape(TP, 1)

    xs_pad = _sc_dispatch(xs, tok_idx, aux)

    y_pad = _gmm(xs_pad, W_fc, b_fc, W_proj, b_proj, w_pad,
                 block_expert, meta, NB)

    out = _sc_combine(y_pad, tok_idx, aux, T)
    return out[:T].reshape(B, S, D)


# speedup vs baseline: 1.4669x; 1.2094x over previous
import functools

import jax
import jax.numpy as jnp
from jax import lax
from jax.experimental import pallas as pl
from jax.experimental.pallas import tpu as pltpu
from jax.experimental.pallas import tpu_sc as plsc

BT = 128
RB = 512
CH = 128
SPREAD = 1024
L = 16


def _sc_mesh():
    return plsc.VectorSubcoreMesh(core_axis_name="c", subcore_axis_name="s")


def _wid():
    info = plsc.get_sparse_core_info()
    return lax.axis_index("s") * info.num_cores + lax.axis_index("c")


def _nw():
    info = plsc.get_sparse_core_info()
    return info.num_cores * info.num_subcores


def _lane_extract(vec, k):
    lanes = lax.iota(jnp.int32, L)
    return jnp.sum(jnp.where(lanes == k, vec, 0))


def _router_body(x_ref, wr_ref, sel_ref, w_ref, cnt_ref):
    j = pl.program_id(0)
    E = wr_ref.shape[0]
    x = x_ref[...]
    wr = wr_ref[...]
    logits = lax.dot_general(x, wr, (((1,), (1,)), ((), ())),
                             preferred_element_type=jnp.float32)
    m = jnp.max(logits, axis=1, keepdims=True)
    s = jnp.sum(jnp.exp(logits - m), axis=1)
    eidx = lax.broadcasted_iota(jnp.int32, logits.shape, 1)
    sel = jnp.min(jnp.where(logits == m, eidx, E), axis=1)
    sel_ref[...] = sel.reshape(1, -1)
    w_ref[...] = (1.0 / s).reshape(1, -1)
    cnt = jnp.sum(jnp.where(sel[:, None] == eidx[:1], 1, 0), axis=0,
                  keepdims=True)

    @pl.when(j == 0)
    def _():
        cnt_ref[...] = cnt

    @pl.when(j > 0)
    def _():
        cnt_ref[...] = cnt_ref[...] + cnt


def _route(xs, W_router):
    T, D = xs.shape
    E = W_router.shape[0]
    nblk = T // RB
    sel, w, counts = pl.pallas_call(
        _router_body,
        grid=(nblk,),
        in_specs=[
            pl.BlockSpec((RB, D), lambda j: (j, 0)),
            pl.BlockSpec((E, D), lambda j: (0, 0)),
        ],
        out_specs=[
            pl.BlockSpec((1, RB), lambda j: (0, j)),
            pl.BlockSpec((1, RB), lambda j: (0, j)),
            pl.BlockSpec((1, E), lambda j: (0, 0)),
        ],
        out_shape=[
            jax.ShapeDtypeStruct((1, T), jnp.int32),
            jax.ShapeDtypeStruct((1, T), jnp.float32),
            jax.ShapeDtypeStruct((1, E), jnp.int32),
        ],
        compiler_params=pltpu.CompilerParams(
            dimension_semantics=("arbitrary",)),
    )(xs, W_router)
    return sel[0], w[0], counts[0]


def _sc_fill(sel, base, w, E, TP):
    T = sel.shape[0]
    NW = _nw()
    EPW = E // NW
    n_vec = T // L
    maxb = T // BT

    @functools.partial(
        pl.kernel, mesh=_sc_mesh(),
        out_type=[
            jax.ShapeDtypeStruct((TP,), jnp.int32),
            jax.ShapeDtypeStruct((TP,), jnp.float32),
        ],
        scratch_types=[
            pltpu.VMEM((T,), jnp.int32),
            pltpu.VMEM((T,), jnp.int32),
            pltpu.VMEM((L,), jnp.int32),
            pltpu.VMEM((T,), jnp.float32),
            pltpu.VMEM((T,), jnp.float32),
        ],
        compiler_params=pltpu.CompilerParams(needs_layout_passes=False),
    )
    def k(sel_hbm, base_hbm, w_hbm, tok_hbm, wpad_hbm,
          sel_v, buf, base_v, w_v, wbuf):
        wid = _wid()
        pltpu.sync_copy(sel_hbm, sel_v)
        pltpu.sync_copy(w_hbm, w_v)
        pltpu.sync_copy(base_hbm.at[pl.ds(wid * 8, 8)], base_v.at[pl.ds(0, 8)])
        lanes = lax.iota(jnp.int32, L)
        bvec = base_v[...]
        for j in range(EPW):
            e = wid * EPW + j

            def pre(i, _):
                buf[pl.ds(i * L, L)] = T + ((i * L + lanes) & (SPREAD - 1))
                wbuf[pl.ds(i * L, L)] = jnp.zeros((L,), jnp.float32)
                return 0

            lax.fori_loop(0, n_vec, pre, 0)

            def scan(i, off):
                v = sel_v[pl.ds(i * L, L)]
                m = v == e
                mi = jnp.where(m, 1, 0)
                cs = plsc.cumsum(mi)
                pos = off + cs - mi
                plsc.store_scatter(buf, [pos], i * L + lanes, mask=m)
                plsc.store_scatter(wbuf, [pos], w_v[pl.ds(i * L, L)], mask=m)
                return off + _lane_extract(cs, L - 1)

            cnt = lax.fori_loop(0, n_vec, scan, 0)
            base_blk = _lane_extract(bvec, j)
            nb = (cnt + BT - 1) // BT
            for b in range(maxb):
                @pl.when(b < nb)
                def _():
                    pltpu.sync_copy(
                        buf.at[pl.ds(b * BT, BT)],
                        tok_hbm.at[pl.ds((base_blk + b) * BT, BT)])
                    pltpu.sync_copy(
                        wbuf.at[pl.ds(b * BT, BT)],
                        wpad_hbm.at[pl.ds((base_blk + b) * BT, BT)])

    return k(sel, base, w)


def _sc_dispatch(xs, tok_idx, aux):
    T, D = xs.shape
    TP = tok_idx.shape[0]
    NW = _nw()
    per_w = TP // NW
    n_ch = per_w // CH

    @functools.partial(
        pl.kernel, mesh=_sc_mesh(),
        out_type=jax.ShapeDtypeStruct((TP, D), jnp.float32),
        scratch_types=[
            pltpu.VMEM((CH,), jnp.int32),
            pltpu.VMEM((CH,), jnp.int32),
            pltpu.VMEM((CH, D), jnp.float32),
            pltpu.VMEM((L,), jnp.int32),
            pltpu.SemaphoreType.DMA,
        ],
        compiler_params=pltpu.CompilerParams(needs_layout_passes=False),
    )
    def k(xs_hbm, idx_hbm, aux_hbm, out_hbm, idx_v, idx2_v, rows_v, lim_v, sem):
        wid = _wid()
        base = wid * per_w
        pltpu.sync_copy(aux_hbm, lim_v.at[pl.ds(0, 8)])
        limit = _lane_extract(lim_v[...], 0)
        for c in range(n_ch):
            off = base + c * CH

            @pl.when(off < limit)
            def _():
                pltpu.sync_copy(idx_hbm.at[pl.ds(off, CH)], idx_v)
                for u in range(CH // L):
                    idx2_v[pl.ds(u * L, L)] = idx_v[pl.ds(u * L, L)] & (T - 1)
                pltpu.async_copy(xs_hbm.at[idx2_v], rows_v, sem).wait()
                pltpu.sync_copy(rows_v, out_hbm.at[pl.ds(off, CH)])

    return k(xs, tok_idx, aux)


def _sc_combine(y_pad, tok_idx, aux, T):
    TP, D = y_pad.shape
    NW = _nw()
    per_w = TP // NW
    n_ch = per_w // CH

    @functools.partial(
        pl.kernel, mesh=_sc_mesh(),
        out_type=jax.ShapeDtypeStruct((T + SPREAD, D), jnp.float32),
        scratch_types=[
            pltpu.VMEM((CH,), jnp.int32),
            pltpu.VMEM((CH, D), jnp.float32),
            pltpu.VMEM((L,), jnp.int32),
            pltpu.SemaphoreType.DMA,
        ],
        compiler_params=pltpu.CompilerParams(needs_layout_passes=False),
    )
    def k(y_hbm, idx_hbm, aux_hbm, out_hbm, idx_v, rows_v, lim_v, sem):
        wid = _wid()
        base = wid * per_w
        pltpu.sync_copy(aux_hbm, lim_v.at[pl.ds(0, 8)])
        limit = _lane_extract(lim_v[...], 0)
        for c in range(n_ch):
            off = base + c * CH

            @pl.when(off < limit)
            def _():
                pltpu.sync_copy(idx_hbm.at[pl.ds(off, CH)], idx_v)
                pltpu.sync_copy(y_hbm.at[pl.ds(off, CH)], rows_v)
                pltpu.async_copy(rows_v, out_hbm.at[idx_v], sem).wait()

    return k(y_pad, tok_idx, aux)


def _gmm_body(be_ref, meta_ref, x_ref, wfc_ref, bfc_ref, wproj_ref,
              bproj_ref, wgt_ref, y_ref):
    j = pl.program_id(0)

    @pl.when(j < meta_ref[0])
    def _():
        x = x_ref[...]
        h = lax.dot_general(x, wfc_ref[0], (((1,), (1,)), ((), ())),
                            preferred_element_type=jnp.float32)
        h = h + bfc_ref[0]
        h = 0.5 * h * (1.0 + lax.erf(h * 0.7071067811865476))
        y = lax.dot_general(h, wproj_ref[0], (((1,), (1,)), ((), ())),
                            preferred_element_type=jnp.float32)
        y = y + bproj_ref[0]
        y_ref[...] = y * wgt_ref[...]


def _gmm(xs_pad, W_fc, b_fc, W_proj, b_proj, w_pad, be, meta, nb):
    TP, D = xs_pad.shape
    E, F, _ = W_fc.shape
    grid_spec = pltpu.PrefetchScalarGridSpec(
        num_scalar_prefetch=2,
        grid=(nb,),
        in_specs=[
            pl.BlockSpec((BT, D), lambda j, be, meta: (j, 0)),
            pl.BlockSpec((1, F, D), lambda j, be, meta: (be[j], 0, 0)),
            pl.BlockSpec((1, 1, F), lambda j, be, meta: (be[j], 0, 0)),
            pl.BlockSpec((1, D, F), lambda j, be, meta: (be[j], 0, 0)),
            pl.BlockSpec((1, 1, D), lambda j, be, meta: (be[j], 0, 0)),
            pl.BlockSpec((BT, 1), lambda j, be, meta: (j, 0)),
        ],
        out_specs=pl.BlockSpec((BT, D), lambda j, be, meta: (j, 0)),
    )
    return pl.pallas_call(
        _gmm_body,
        grid_spec=grid_spec,
        out_shape=jax.ShapeDtypeStruct((TP, D), jnp.float32),
        compiler_params=pltpu.CompilerParams(
            dimension_semantics=("arbitrary",)),
    )(be, meta, xs_pad, W_fc, b_fc.reshape(E, 1, F), W_proj,
      b_proj.reshape(E, 1, D), w_pad)


def kernel(x, W_router, W_fc, b_fc, W_proj, b_proj):
    B, S, D = x.shape
    E, F, _ = W_fc.shape
    T = B * S
    NB = T // BT + E
    TP = NB * BT
    NW = 32
    EPW = E // NW

    xs = x.reshape(T, D)
    sel, w, counts = _route(xs, W_router)

    nblk = (counts + BT - 1) // BT
    blk_end = jnp.cumsum(nblk).astype(jnp.int32)
    nba = blk_end[E - 1]
    blk_start = blk_end - nblk
    base8 = jnp.zeros((NW, 8), jnp.int32).at[:, :EPW].set(
        blk_start.reshape(NW, EPW)).reshape(NW * 8)
    aux = jnp.full((8,), nba * BT, jnp.int32)
    jblk = jnp.minimum(jnp.arange(NB, dtype=jnp.int32), nba - 1)
    block_expert = jnp.searchsorted(blk_end, jblk, side="right").astype(jnp.int32)
    meta = nba.reshape(1)

    tok_idx, w_pad = _sc_fill(sel, base8, w, E, TP)
    w_pad = w_pad.reshape(TP, 1)

    xs_pad = _sc_dispatch(xs, tok_idx, aux)

    y_pad = _gmm(xs_pad, W_fc, b_fc, W_proj, b_proj, w_pad,
                 block_expert, meta, NB)

    out = _sc_combine(y_pad, tok_idx, aux, T)
    return out[:T].reshape(B, S, D)
